# Initial kernel scaffold; baseline (speedup 1.0000x reference)
import jax
import jax.numpy as jnp
from jax.experimental import pallas as pl


def kernel(x, edge_index, W1, b1, W2, b2):
    raise NotImplementedError("write your pallas kernel here")


# speedup vs baseline: 13.5860x; 13.5860x over previous
import functools

import jax
import jax.numpy as jnp
from jax import lax
from jax.experimental import pallas as pl
from jax.experimental.pallas import tpu as pltpu
from jax.experimental.pallas import tpu_sc as plsc

N = 10000
D = 128
E = 320000
NPAD = 10240
NC, NS = 2, 16
NW = NC * NS
EPW = E // NW
CHUNK = 80
NCHUNK = EPW // CHUNK
ROWS_PER_TILE = NPAD // NS

_mesh = plsc.VectorSubcoreMesh(core_axis_name="c", subcore_axis_name="s")


@functools.partial(
    pl.kernel,
    out_type=jax.ShapeDtypeStruct((NC, NPAD), jnp.float32),
    mesh=_mesh,
    scratch_types=[
        pltpu.VMEM((CHUNK,), jnp.int32),
        pltpu.VMEM((CHUNK,), jnp.float32),
        pltpu.VMEM((ROWS_PER_TILE,), jnp.float32),
        pltpu.VMEM_SHARED((NPAD,), jnp.float32),
    ],
)
def _deg_kernel(dst_hbm, z1_hbm, out_hbm, dstv, onesv, zv, acc):
    c = lax.axis_index("c")
    s = lax.axis_index("s")
    for j in range(CHUNK // 16):
        onesv[pl.ds(16 * j, 16)] = jnp.ones((16,), jnp.float32)
    pltpu.sync_copy(z1_hbm, zv)
    pltpu.sync_copy(zv, acc.at[pl.ds(s * ROWS_PER_TILE, ROWS_PER_TILE)])
    plsc.subcore_barrier()

    base_w = (c * NS + s) * EPW

    def step(i, _):
        base = pl.multiple_of(base_w + i * CHUNK, 8)
        pltpu.sync_copy(dst_hbm.at[pl.ds(base, CHUNK)], dstv)
        pltpu.sync_copy(onesv, acc.at[dstv], add=True)
        return ()

    lax.fori_loop(0, NCHUNK, step, ())
    plsc.subcore_barrier()
    r0 = s * ROWS_PER_TILE
    pltpu.sync_copy(acc.at[pl.ds(r0, ROWS_PER_TILE)],
                    out_hbm.at[c, pl.ds(r0, ROWS_PER_TILE)])


@functools.partial(
    pl.kernel,
    out_type=jax.ShapeDtypeStruct((NC, NPAD, D), jnp.float32),
    mesh=_mesh,
    scratch_types=[
        pltpu.VMEM((CHUNK,), jnp.int32),
        pltpu.VMEM((CHUNK,), jnp.int32),
        pltpu.VMEM((CHUNK, D), jnp.float32),
        pltpu.VMEM((128, D), jnp.float32),
        pltpu.VMEM_SHARED((NPAD, D), jnp.float32),
        pltpu.SemaphoreType.DMA,
    ],
)
def _scatter_kernel(g_hbm, src_hbm, dst_hbm, z2_hbm, out_hbm,
                    srcv, dstv, rows, zb, acc, sem):
    c = lax.axis_index("c")
    s = lax.axis_index("s")
    pltpu.sync_copy(z2_hbm, zb)
    for j in range(ROWS_PER_TILE // 128):
        pltpu.sync_copy(zb, acc.at[pl.ds(s * ROWS_PER_TILE + j * 128, 128), :])
    plsc.subcore_barrier()

    base_w = (c * NS + s) * EPW

    def step(i, _):
        base = pl.multiple_of(base_w + i * CHUNK, 8)
        pltpu.sync_copy(src_hbm.at[pl.ds(base, CHUNK)], srcv)
        pltpu.async_copy(g_hbm.at[srcv], rows, sem).wait()
        pltpu.sync_copy(dst_hbm.at[pl.ds(base, CHUNK)], dstv)
        pltpu.sync_copy(rows, acc.at[dstv], add=True)
        return ()

    lax.fori_loop(0, NCHUNK, step, ())
    plsc.subcore_barrier()
    for j in range(ROWS_PER_TILE // 128):
        r0 = s * ROWS_PER_TILE + j * 128
        pltpu.sync_copy(acc.at[pl.ds(r0, 128), :],
                        out_hbm.at[c, pl.ds(r0, 128), :])


_RB = 1024
_GRID = (N + _RB - 1) // _RB


def _dense1_body(x_ref, w_ref, d0_ref, d1_ref, g_ref):
    dis = lax.rsqrt(d0_ref[...] + d1_ref[...] + 1.0)
    h = jnp.dot(x_ref[...], w_ref[...], preferred_element_type=jnp.float32)
    g_ref[...] = h * dis[:, None]


def _dense2_body(a_ref, g_ref, d0_ref, d1_ref, b_ref, w_ref, out_ref):
    dis = lax.rsqrt(d0_ref[...] + d1_ref[...] + 1.0)
    z = dis[:, None] * (a_ref[0] + a_ref[1] + g_ref[...]) + b_ref[...][None, :]
    z = jnp.maximum(z, 0.0)
    out_ref[...] = jnp.dot(z, w_ref[...],
                           preferred_element_type=jnp.float32) * dis[:, None]


def _dense3_body(a_ref, g_ref, d0_ref, d1_ref, b_ref, out_ref):
    dis = lax.rsqrt(d0_ref[...] + d1_ref[...] + 1.0)
    out_ref[...] = (dis[:, None] * (a_ref[0] + a_ref[1] + g_ref[...])
                    + b_ref[...][None, :])


_row_spec = pl.BlockSpec((_RB, D), lambda i: (i, 0))
_deg_spec = pl.BlockSpec((_RB,), lambda i: (i,))
_a_spec = pl.BlockSpec((NC, _RB, D), lambda i: (0, i, 0))
_w_spec = pl.BlockSpec((D, D), lambda i: (0, 0))
_b_spec = pl.BlockSpec((D,), lambda i: (0,))

_dense1 = pl.pallas_call(
    _dense1_body,
    grid=_GRID,
    in_specs=[_row_spec, _w_spec, _deg_spec, _deg_spec],
    out_specs=_row_spec,
    out_shape=jax.ShapeDtypeStruct((N, D), jnp.float32),
)

_dense2 = pl.pallas_call(
    _dense2_body,
    grid=_GRID,
    in_specs=[_a_spec, _row_spec, _deg_spec, _deg_spec, _b_spec, _w_spec],
    out_specs=_row_spec,
    out_shape=jax.ShapeDtypeStruct((N, D), jnp.float32),
)

_dense3 = pl.pallas_call(
    _dense3_body,
    grid=_GRID,
    in_specs=[_a_spec, _row_spec, _deg_spec, _deg_spec, _b_spec],
    out_specs=_row_spec,
    out_shape=jax.ShapeDtypeStruct((N, D), jnp.float32),
)


def kernel(x, edge_index, W1, b1, W2, b2):
    src = edge_index[0].astype(jnp.int32)
    dst = edge_index[1].astype(jnp.int32)
    z1 = jnp.zeros((ROWS_PER_TILE,), jnp.float32)
    z2 = jnp.zeros((128, D), jnp.float32)

    degs = _deg_kernel(dst, z1)
    d0, d1 = degs[0], degs[1]

    g1 = _dense1(x, W1, d0, d1)
    a1 = _scatter_kernel(g1, src, dst, z2)
    g2 = _dense2(a1, g1, d0, d1, b1, W2)
    a2 = _scatter_kernel(g2, src, dst, z2)
    out = _dense3(a2, g2, d0, d1, b2)
    return out
